# Initial kernel scaffold; baseline (speedup 1.0000x reference)
import jax
import jax.numpy as jnp
from jax.experimental import pallas as pl


def kernel(x, W, a_src, a_dst, ln_scale, ln_bias):
    raise NotImplementedError("write your pallas kernel here")


# speedup vs baseline: 1.8864x; 1.8864x over previous
import functools

import jax
import jax.numpy as jnp
from jax.experimental import pallas as pl
from jax.experimental.pallas import tpu as pltpu

NUM_HEADS = 4
OUT_FEATURES = 32
IN_FEATURES = 128
HD = NUM_HEADS * OUT_FEATURES


def _proj_kernel(x_ref, w_ref, asrc_ref, adst_ref, h_ref, ei_ref, ejT_ref, m_ref):
    x = x_ref[0]
    h = jnp.dot(x, w_ref[...], preferred_element_type=jnp.float32)
    h_ref[0] = h
    ei_ref[0] = jnp.dot(h, asrc_ref[...], preferred_element_type=jnp.float32)
    ejT = jax.lax.dot_general(
        adst_ref[...], h,
        dimension_numbers=(((0,), (1,)), ((), ())),
        preferred_element_type=jnp.float32,
    )
    ejT_ref[0] = ejT
    m_ref[0] = jnp.max(ejT, axis=1, keepdims=True).T


def _attn_kernel(ei_ref, ejT_ref, h_ref, m_ref, x_ref, lns_ref, lnb_ref, out_ref):
    ei = ei_ref[0]
    ejT = ejT_ref[0]
    hf = h_ref[0]
    Mv = m_ref[0]
    outs = []
    for hh in range(NUM_HEADS):
        c = ei[:, hh:hh + 1]
        mrow = c + Mv[:, hh:hh + 1]
        m = jnp.maximum(mrow, 0.2 * mrow)
        t = c + ejT[hh:hh + 1, :]
        t = jnp.maximum(t, 0.2 * t)
        p = jnp.exp(t - m)
        s = jnp.dot(p, hf[:, hh * OUT_FEATURES:(hh + 1) * OUT_FEATURES],
                    preferred_element_type=jnp.float32)
        z = jnp.sum(p, axis=1, keepdims=True)
        outs.append(s / z)
    hp = jnp.concatenate(outs, axis=1) + x_ref[0]
    mean = jnp.mean(hp, axis=1, keepdims=True)
    ctr = hp - mean
    var = jnp.mean(ctr * ctr, axis=1, keepdims=True)
    out_ref[0] = ctr * jax.lax.rsqrt(var + 1e-5) * lns_ref[...] + lnb_ref[...]


@functools.partial(jax.jit, static_argnames=())
def kernel(x, W, a_src, a_dst, ln_scale, ln_bias):
    B, N, IN = x.shape
    H, D = a_src.shape
    eye = jnp.eye(H, dtype=x.dtype)
    A_src = (a_src[:, :, None] * eye[:, None, :]).reshape(H * D, H)
    A_dst = (a_dst[:, :, None] * eye[:, None, :]).reshape(H * D, H)

    h, ei, ejT, M = pl.pallas_call(
        _proj_kernel,
        grid=(B,),
        in_specs=[
            pl.BlockSpec((1, N, IN), lambda b: (b, 0, 0)),
            pl.BlockSpec((IN, H * D), lambda b: (0, 0)),
            pl.BlockSpec((H * D, H), lambda b: (0, 0)),
            pl.BlockSpec((H * D, H), lambda b: (0, 0)),
        ],
        out_specs=[
            pl.BlockSpec((1, N, H * D), lambda b: (b, 0, 0)),
            pl.BlockSpec((1, N, H), lambda b: (b, 0, 0)),
            pl.BlockSpec((1, H, N), lambda b: (b, 0, 0)),
            pl.BlockSpec((1, 1, H), lambda b: (b, 0, 0)),
        ],
        out_shape=[
            jax.ShapeDtypeStruct((B, N, H * D), jnp.float32),
            jax.ShapeDtypeStruct((B, N, H), jnp.float32),
            jax.ShapeDtypeStruct((B, H, N), jnp.float32),
            jax.ShapeDtypeStruct((B, 1, H), jnp.float32),
        ],
        compiler_params=pltpu.CompilerParams(
            dimension_semantics=("parallel",),
        ),
    )(x, W, A_src, A_dst)

    BN = 256
    out = pl.pallas_call(
        _attn_kernel,
        grid=(B, N // BN),
        in_specs=[
            pl.BlockSpec((1, BN, H), lambda b, nb: (b, nb, 0)),
            pl.BlockSpec((1, H, N), lambda b, nb: (b, 0, 0)),
            pl.BlockSpec((1, N, H * D), lambda b, nb: (b, 0, 0)),
            pl.BlockSpec((1, 1, H), lambda b, nb: (b, 0, 0)),
            pl.BlockSpec((1, BN, IN), lambda b, nb: (b, nb, 0)),
            pl.BlockSpec((1, H * D), lambda b, nb: (0, 0)),
            pl.BlockSpec((1, H * D), lambda b, nb: (0, 0)),
        ],
        out_specs=pl.BlockSpec((1, BN, H * D), lambda b, nb: (b, nb, 0)),
        out_shape=jax.ShapeDtypeStruct((B, N, H * D), jnp.float32),
        compiler_params=pltpu.CompilerParams(
            dimension_semantics=("parallel", "parallel"),
        ),
    )(ei, ejT, h, M, x, ln_scale.reshape(1, H * D), ln_bias.reshape(1, H * D))
    return out


# speedup vs baseline: 2.1926x; 1.1623x over previous
import functools

import jax
import jax.numpy as jnp
import numpy as np
from jax.experimental import pallas as pl
from jax.experimental.pallas import tpu as pltpu

NUM_HEADS = 4
OUT_FEATURES = 32
IN_FEATURES = 128
HD = NUM_HEADS * OUT_FEATURES
SLAB = 128


def _proj_kernel(x_ref, w_ref, waug_ref, asrc_ref, adst_ref,
                 haug_ref, ei_ref, ejT_ref, m_ref):
    x = x_ref[0]
    h = jnp.dot(x, w_ref[...], preferred_element_type=jnp.float32)
    hs = jnp.dot(x, waug_ref[...], preferred_element_type=jnp.float32)
    n = x.shape[0]
    for hh in range(NUM_HEADS):
        haug_ref[0, hh, :, :] = hs[:, hh * SLAB:(hh + 1) * SLAB]
        haug_ref[0, hh, :, OUT_FEATURES:OUT_FEATURES + 1] = jnp.ones(
            (n, 1), jnp.float32)
    ei_ref[0] = jnp.dot(h, asrc_ref[...], preferred_element_type=jnp.float32)
    ejT = jax.lax.dot_general(
        adst_ref[...], h,
        dimension_numbers=(((0,), (1,)), ((), ())),
        preferred_element_type=jnp.float32,
    )
    ejT_ref[0] = ejT
    m_ref[0] = jnp.max(ejT, axis=1, keepdims=True).T


def _attn_kernel(ei_ref, ejT_ref, haug_ref, m_ref, x_ref, lns_ref, lnb_ref,
                 out_ref):
    ei = ei_ref[0]
    ejT = ejT_ref[0]
    Mv = m_ref[0]
    outs = []
    for hh in range(NUM_HEADS):
        c = ei[:, hh:hh + 1]
        mrow = c + Mv[:, hh:hh + 1]
        m = jnp.maximum(mrow, 0.2 * mrow)
        t = c + ejT[hh:hh + 1, :]
        t = jnp.maximum(t, 0.2 * t)
        p = jnp.exp2(t - m)
        sz = jnp.dot(p, haug_ref[0, hh],
                     preferred_element_type=jnp.float32)
        outs.append(sz[:, :OUT_FEATURES] / sz[:, OUT_FEATURES:OUT_FEATURES + 1])
    hp = jnp.concatenate(outs, axis=1) + x_ref[0]
    mean = jnp.mean(hp, axis=1, keepdims=True)
    ctr = hp - mean
    var = jnp.mean(ctr * ctr, axis=1, keepdims=True)
    out_ref[0] = ctr * jax.lax.rsqrt(var + 1e-5) * lns_ref[...] + lnb_ref[...]


@functools.partial(jax.jit, static_argnames=())
def kernel(x, W, a_src, a_dst, ln_scale, ln_bias):
    B, N, IN = x.shape
    H, D = a_src.shape
    LOG2E = np.float32(np.log2(np.e))
    eye = jnp.eye(H, dtype=x.dtype)
    A_src = (LOG2E * a_src[:, :, None] * eye[:, None, :]).reshape(H * D, H)
    A_dst = (LOG2E * a_dst[:, :, None] * eye[:, None, :]).reshape(H * D, H)
    W_aug = jnp.zeros((IN, H * SLAB), jnp.float32)
    for hh in range(H):
        W_aug = W_aug.at[:, hh * SLAB:hh * SLAB + D].set(
            W[:, hh * D:(hh + 1) * D])

    haug, ei, ejT, M = pl.pallas_call(
        _proj_kernel,
        grid=(B,),
        in_specs=[
            pl.BlockSpec((1, N, IN), lambda b: (b, 0, 0)),
            pl.BlockSpec((IN, H * D), lambda b: (0, 0)),
            pl.BlockSpec((IN, H * SLAB), lambda b: (0, 0)),
            pl.BlockSpec((H * D, H), lambda b: (0, 0)),
            pl.BlockSpec((H * D, H), lambda b: (0, 0)),
        ],
        out_specs=[
            pl.BlockSpec((1, H, N, SLAB), lambda b: (b, 0, 0, 0)),
            pl.BlockSpec((1, N, H), lambda b: (b, 0, 0)),
            pl.BlockSpec((1, H, N), lambda b: (b, 0, 0)),
            pl.BlockSpec((1, 1, H), lambda b: (b, 0, 0)),
        ],
        out_shape=[
            jax.ShapeDtypeStruct((B, H, N, SLAB), jnp.float32),
            jax.ShapeDtypeStruct((B, N, H), jnp.float32),
            jax.ShapeDtypeStruct((B, H, N), jnp.float32),
            jax.ShapeDtypeStruct((B, 1, H), jnp.float32),
        ],
        compiler_params=pltpu.CompilerParams(
            dimension_semantics=("parallel",),
        ),
    )(x, W, W_aug, A_src, A_dst)

    BN = 256
    out = pl.pallas_call(
        _attn_kernel,
        grid=(B, N // BN),
        in_specs=[
            pl.BlockSpec((1, BN, H), lambda b, nb: (b, nb, 0)),
            pl.BlockSpec((1, H, N), lambda b, nb: (b, 0, 0)),
            pl.BlockSpec((1, H, N, SLAB), lambda b, nb: (b, 0, 0, 0)),
            pl.BlockSpec((1, 1, H), lambda b, nb: (b, 0, 0)),
            pl.BlockSpec((1, BN, IN), lambda b, nb: (b, nb, 0)),
            pl.BlockSpec((1, HD), lambda b, nb: (0, 0)),
            pl.BlockSpec((1, HD), lambda b, nb: (0, 0)),
        ],
        out_specs=pl.BlockSpec((1, BN, HD), lambda b, nb: (b, nb, 0)),
        out_shape=jax.ShapeDtypeStruct((B, N, HD), jnp.float32),
        compiler_params=pltpu.CompilerParams(
            dimension_semantics=("parallel", "parallel"),
        ),
    )(ei, ejT, haug, M, x, ln_scale.reshape(1, HD), ln_bias.reshape(1, HD))
    return out


# speedup vs baseline: 2.3586x; 1.0757x over previous
import functools

import jax
import jax.numpy as jnp
import numpy as np
from jax.experimental import pallas as pl
from jax.experimental.pallas import tpu as pltpu

NUM_HEADS = 4
OUT_FEATURES = 32
IN_FEATURES = 128
HD = NUM_HEADS * OUT_FEATURES
SLAB = 128


def _proj_kernel(x_ref, w_ref, waug_ref, asrc_ref, adst_ref,
                 haug_ref, ei_ref, ejT_ref, m_ref):
    x = x_ref[0]
    h = jnp.dot(x, w_ref[...], preferred_element_type=jnp.float32)
    hs = jnp.dot(x, waug_ref[...], preferred_element_type=jnp.float32)
    n = x.shape[0]
    for hh in range(NUM_HEADS):
        haug_ref[0, hh, :, :] = hs[:, hh * SLAB:(hh + 1) * SLAB].astype(
            jnp.bfloat16)
        haug_ref[0, hh, :, OUT_FEATURES:OUT_FEATURES + 1] = jnp.ones(
            (n, 1), jnp.bfloat16)
    ei_ref[0] = jnp.dot(h, asrc_ref[...], preferred_element_type=jnp.float32)
    ejT = jax.lax.dot_general(
        adst_ref[...], h,
        dimension_numbers=(((0,), (1,)), ((), ())),
        preferred_element_type=jnp.float32,
    )
    ejT_ref[0] = ejT
    m_ref[0] = jnp.max(ejT, axis=1, keepdims=True).T


def _attn_kernel(ei_ref, ejT_ref, haug_ref, m_ref, x_ref, lns_ref, lnb_ref,
                 out_ref):
    ei = ei_ref[0]
    ejT = ejT_ref[0]
    Mv = m_ref[0]
    outs = []
    for hh in range(NUM_HEADS):
        c = ei[:, hh:hh + 1]
        mrow = c + Mv[:, hh:hh + 1]
        m = jnp.maximum(mrow, 0.2 * mrow)
        a1 = (c - m) + ejT[hh:hh + 1, :]
        a2 = (0.2 * c - m) + 0.2 * ejT[hh:hh + 1, :]
        p = jnp.exp2(jnp.maximum(a1, a2)).astype(jnp.bfloat16)
        sz = jnp.dot(p, haug_ref[0, hh],
                     preferred_element_type=jnp.float32)
        outs.append(sz[:, :OUT_FEATURES] / sz[:, OUT_FEATURES:OUT_FEATURES + 1])
    hp = jnp.concatenate(outs, axis=1) + x_ref[0]
    mean = jnp.mean(hp, axis=1, keepdims=True)
    ctr = hp - mean
    var = jnp.mean(ctr * ctr, axis=1, keepdims=True)
    out_ref[0] = ctr * jax.lax.rsqrt(var + 1e-5) * lns_ref[...] + lnb_ref[...]


@functools.partial(jax.jit, static_argnames=())
def kernel(x, W, a_src, a_dst, ln_scale, ln_bias):
    B, N, IN = x.shape
    H, D = a_src.shape
    LOG2E = np.float32(np.log2(np.e))
    eye = jnp.eye(H, dtype=x.dtype)
    A_src = (LOG2E * a_src[:, :, None] * eye[:, None, :]).reshape(H * D, H)
    A_dst = (LOG2E * a_dst[:, :, None] * eye[:, None, :]).reshape(H * D, H)
    W_aug = jnp.zeros((IN, H * SLAB), jnp.float32)
    for hh in range(H):
        W_aug = W_aug.at[:, hh * SLAB:hh * SLAB + D].set(
            W[:, hh * D:(hh + 1) * D])

    haug, ei, ejT, M = pl.pallas_call(
        _proj_kernel,
        grid=(B,),
        in_specs=[
            pl.BlockSpec((1, N, IN), lambda b: (b, 0, 0)),
            pl.BlockSpec((IN, H * D), lambda b: (0, 0)),
            pl.BlockSpec((IN, H * SLAB), lambda b: (0, 0)),
            pl.BlockSpec((H * D, H), lambda b: (0, 0)),
            pl.BlockSpec((H * D, H), lambda b: (0, 0)),
        ],
        out_specs=[
            pl.BlockSpec((1, H, N, SLAB), lambda b: (b, 0, 0, 0)),
            pl.BlockSpec((1, N, H), lambda b: (b, 0, 0)),
            pl.BlockSpec((1, H, N), lambda b: (b, 0, 0)),
            pl.BlockSpec((1, 1, H), lambda b: (b, 0, 0)),
        ],
        out_shape=[
            jax.ShapeDtypeStruct((B, H, N, SLAB), jnp.bfloat16),
            jax.ShapeDtypeStruct((B, N, H), jnp.float32),
            jax.ShapeDtypeStruct((B, H, N), jnp.float32),
            jax.ShapeDtypeStruct((B, 1, H), jnp.float32),
        ],
        compiler_params=pltpu.CompilerParams(
            dimension_semantics=("parallel",),
        ),
    )(x, W, W_aug, A_src, A_dst)

    BN = 512
    out = pl.pallas_call(
        _attn_kernel,
        grid=(B, N // BN),
        in_specs=[
            pl.BlockSpec((1, BN, H), lambda b, nb: (b, nb, 0)),
            pl.BlockSpec((1, H, N), lambda b, nb: (b, 0, 0)),
            pl.BlockSpec((1, H, N, SLAB), lambda b, nb: (b, 0, 0, 0)),
            pl.BlockSpec((1, 1, H), lambda b, nb: (b, 0, 0)),
            pl.BlockSpec((1, BN, IN), lambda b, nb: (b, nb, 0)),
            pl.BlockSpec((1, HD), lambda b, nb: (0, 0)),
            pl.BlockSpec((1, HD), lambda b, nb: (0, 0)),
        ],
        out_specs=pl.BlockSpec((1, BN, HD), lambda b, nb: (b, nb, 0)),
        out_shape=jax.ShapeDtypeStruct((B, N, HD), jnp.float32),
        compiler_params=pltpu.CompilerParams(
            dimension_semantics=("parallel", "parallel"),
        ),
    )(ei, ejT, haug, M, x, ln_scale.reshape(1, HD), ln_bias.reshape(1, HD))
    return out


# speedup vs baseline: 2.4645x; 1.0449x over previous
import functools

import jax
import jax.numpy as jnp
import numpy as np
from jax.experimental import pallas as pl
from jax.experimental.pallas import tpu as pltpu

NUM_HEADS = 4
OUT_FEATURES = 32
IN_FEATURES = 128
HD = NUM_HEADS * OUT_FEATURES
SLAB = 128


def _proj_kernel(x_ref, w_ref, waug_ref, asrc_ref, adst_ref,
                 haug_ref, ei_ref, uv_ref, m_ref):
    x = x_ref[0]
    h = jnp.dot(x, w_ref[...], preferred_element_type=jnp.float32)
    hs = jnp.dot(x, waug_ref[...], preferred_element_type=jnp.float32)
    n = x.shape[0]
    for hh in range(NUM_HEADS):
        haug_ref[0, hh, :, :] = hs[:, hh * SLAB:(hh + 1) * SLAB].astype(
            jnp.bfloat16)
        haug_ref[0, hh, :, OUT_FEATURES:OUT_FEATURES + 1] = jnp.ones(
            (n, 1), jnp.bfloat16)
    ei_ref[0] = jnp.dot(h, asrc_ref[...], preferred_element_type=jnp.float32)
    ejT = jax.lax.dot_general(
        adst_ref[...], h,
        dimension_numbers=(((0,), (1,)), ((), ())),
        preferred_element_type=jnp.float32,
    )
    uv_ref[0, 0] = jnp.exp2(ejT)
    uv_ref[0, 1] = jnp.exp2(0.2 * ejT)
    m_ref[0] = jnp.max(ejT, axis=1, keepdims=True).T


def _attn_kernel(ei_ref, uv_ref, haug_ref, m_ref, x_ref, lns_ref, lnb_ref,
                 out_ref):
    ei = ei_ref[0]
    Mv = m_ref[0]
    outs = []
    for hh in range(NUM_HEADS):
        c = ei[:, hh:hh + 1]
        mrow = c + Mv[:, hh:hh + 1]
        m = jnp.maximum(mrow, 0.2 * mrow)
        U = jnp.exp2(c - m)
        V = jnp.exp2(0.2 * c - m)
        u = uv_ref[0, 0, hh:hh + 1, :]
        v = uv_ref[0, 1, hh:hh + 1, :]
        p = jnp.maximum(U * u, V * v).astype(jnp.bfloat16)
        sz = jnp.dot(p, haug_ref[0, hh],
                     preferred_element_type=jnp.float32)
        outs.append(sz[:, :OUT_FEATURES] / sz[:, OUT_FEATURES:OUT_FEATURES + 1])
    hp = jnp.concatenate(outs, axis=1) + x_ref[0]
    mean = jnp.mean(hp, axis=1, keepdims=True)
    ctr = hp - mean
    var = jnp.mean(ctr * ctr, axis=1, keepdims=True)
    out_ref[0] = ctr * jax.lax.rsqrt(var + 1e-5) * lns_ref[...] + lnb_ref[...]


@functools.partial(jax.jit, static_argnames=())
def kernel(x, W, a_src, a_dst, ln_scale, ln_bias):
    B, N, IN = x.shape
    H, D = a_src.shape
    LOG2E = np.float32(np.log2(np.e))
    eye = jnp.eye(H, dtype=x.dtype)
    A_src = (LOG2E * a_src[:, :, None] * eye[:, None, :]).reshape(H * D, H)
    A_dst = (LOG2E * a_dst[:, :, None] * eye[:, None, :]).reshape(H * D, H)
    W_aug = jnp.zeros((IN, H * SLAB), jnp.float32)
    for hh in range(H):
        W_aug = W_aug.at[:, hh * SLAB:hh * SLAB + D].set(
            W[:, hh * D:(hh + 1) * D])

    haug, ei, uv, M = pl.pallas_call(
        _proj_kernel,
        grid=(B,),
        in_specs=[
            pl.BlockSpec((1, N, IN), lambda b: (b, 0, 0)),
            pl.BlockSpec((IN, H * D), lambda b: (0, 0)),
            pl.BlockSpec((IN, H * SLAB), lambda b: (0, 0)),
            pl.BlockSpec((H * D, H), lambda b: (0, 0)),
            pl.BlockSpec((H * D, H), lambda b: (0, 0)),
        ],
        out_specs=[
            pl.BlockSpec((1, H, N, SLAB), lambda b: (b, 0, 0, 0)),
            pl.BlockSpec((1, N, H), lambda b: (b, 0, 0)),
            pl.BlockSpec((1, 2, H, N), lambda b: (b, 0, 0, 0)),
            pl.BlockSpec((1, 1, H), lambda b: (b, 0, 0)),
        ],
        out_shape=[
            jax.ShapeDtypeStruct((B, H, N, SLAB), jnp.bfloat16),
            jax.ShapeDtypeStruct((B, N, H), jnp.float32),
            jax.ShapeDtypeStruct((B, 2, H, N), jnp.float32),
            jax.ShapeDtypeStruct((B, 1, H), jnp.float32),
        ],
        compiler_params=pltpu.CompilerParams(
            dimension_semantics=("parallel",),
        ),
    )(x, W, W_aug, A_src, A_dst)

    BN = 512
    out = pl.pallas_call(
        _attn_kernel,
        grid=(B, N // BN),
        in_specs=[
            pl.BlockSpec((1, BN, H), lambda b, nb: (b, nb, 0)),
            pl.BlockSpec((1, 2, H, N), lambda b, nb: (b, 0, 0, 0)),
            pl.BlockSpec((1, H, N, SLAB), lambda b, nb: (b, 0, 0, 0)),
            pl.BlockSpec((1, 1, H), lambda b, nb: (b, 0, 0)),
            pl.BlockSpec((1, BN, IN), lambda b, nb: (b, nb, 0)),
            pl.BlockSpec((1, HD), lambda b, nb: (0, 0)),
            pl.BlockSpec((1, HD), lambda b, nb: (0, 0)),
        ],
        out_specs=pl.BlockSpec((1, BN, HD), lambda b, nb: (b, nb, 0)),
        out_shape=jax.ShapeDtypeStruct((B, N, HD), jnp.float32),
        compiler_params=pltpu.CompilerParams(
            dimension_semantics=("parallel", "parallel"),
        ),
    )(ei, uv, haug, M, x, ln_scale.reshape(1, HD), ln_bias.reshape(1, HD))
    return out
